# baseline (speedup 1.0000x reference)
import functools

import jax
import jax.numpy as jnp
from jax import lax
from jax.experimental import pallas as pl
from jax.experimental.pallas import tpu as pltpu
from jax.experimental.pallas import tpu_sc as plsc

N = 10000
E = 320000
D_IN = 128
HID = 512
H1 = 8
D1 = 64
OUT = 2

NC = 2
NS = 16
NW = NC * NS
EW = E // NW
CE = 80
NCH = EW // CE
CEB = 40
NCHB = EW // CEB
NP = 10112
RPS = NP // NS
FCH = 8
FW = HID // FCH
RB = 1000

_mesh = plsc.VectorSubcoreMesh(core_axis_name="c", subcore_axis_name="s")

_GDN = lax.GatherDimensionNumbers(offset_dims=(), collapsed_slice_dims=(0,),
                                  start_index_map=(0,))


def _lane_bcast(v, lane):
    idx = jnp.full((16, 1), lane, jnp.int32)
    return lax.gather(v, idx, _GDN, (1,),
                      mode=lax.GatherScatterMode.PROMISE_IN_BOUNDS)


def _lane_rot(v, k):
    idx = ((lax.iota(jnp.int32, 16) + k) & 15).reshape(16, 1)
    return lax.gather(v, idx, _GDN, (1,),
                      mode=lax.GatherScatterMode.PROMISE_IN_BOUNDS)


def _sum16(v):
    for k in (8, 4, 2, 1):
        v = v + _lane_rot(v, k)
    return v


def _proj_body(x_ref, w_ref, k_ref, q_ref, *v_refs):
    y = jnp.dot(x_ref[...], w_ref[...], preferred_element_type=jnp.float32)
    k_ref[...] = y[:, :HID]
    q_ref[...] = y[:, HID:2 * HID]
    for c, vr in enumerate(v_refs):
        vr[...] = y[:, 2 * HID + c * FW:2 * HID + (c + 1) * FW]


def _proj(x, wcat):
    return pl.pallas_call(
        _proj_body,
        grid=(N // RB,),
        in_specs=[pl.BlockSpec((RB, D_IN), lambda i: (i, 0)),
                  pl.BlockSpec((D_IN, 3 * HID), lambda i: (0, 0))],
        out_specs=[pl.BlockSpec((RB, HID), lambda i: (i, 0)),
                   pl.BlockSpec((RB, HID), lambda i: (i, 0))] +
                  [pl.BlockSpec((RB, FW), lambda i: (i, 0))] * FCH,
        out_shape=[jax.ShapeDtypeStruct((N, HID), jnp.float32)] * 2 +
                  [jax.ShapeDtypeStruct((N, FW), jnp.float32)] * FCH,
    )(x, wcat)


def _kb_body(k_h, q_h, src3_h, dst3_h, ext_h, den_h,
             srcb, dstb, kr0, kr1, qr0, qr1, exr0, exr1, ext0, ext1, cs, zbuf,
             den_sp, semk0, semk1, semq0, semq1, wse0, wse1, wsd0, wsd1):
    c = lax.axis_index("c")
    s = lax.axis_index("s")
    wid = s * NC + c
    iota = lax.iota(jnp.int32, 16)
    gidx = jnp.minimum(iota, H1 - 1) * 16 + 15
    krs = (kr0, kr1)
    qrs = (qr0, qr1)
    exrs = (exr0, exr1)
    exts = (ext0, ext1)
    semks = (semk0, semk1)
    semqs = (semq0, semq1)
    wses = (wse0, wse1)
    wsds = (wsd0, wsd1)

    def zrow(i, carry):
        zbuf[i, :] = jnp.zeros((16,), jnp.float32)
        return carry
    lax.fori_loop(0, RPS, zrow, 0)
    pltpu.sync_copy(zbuf, den_sp.at[pl.ds(s * RPS, RPS)])
    pltpu.sync_copy(src3_h.at[wid], srcb)
    pltpu.sync_copy(dst3_h.at[wid], dstb)
    plsc.subcore_barrier()

    pltpu.async_copy(k_h.at[srcb.at[0]], kr0, semk0)
    pltpu.async_copy(q_h.at[dstb.at[0]], qr0, semq0)

    def half(b, g):
        kr, qr, exr, ext = krs[b], qrs[b], exrs[b], exts[b]
        nb = 1 - b
        gp = jnp.minimum(g + 1, NCHB - 1)
        pltpu.async_copy(k_h.at[srcb.at[gp]], krs[nb], semks[nb])
        pltpu.async_copy(q_h.at[dstb.at[gp]], qrs[nb], semqs[nb])
        pltpu.make_async_copy(k_h.at[srcb.at[g]], kr, semks[b]).wait()
        pltpu.make_async_copy(q_h.at[dstb.at[g]], qr, semqs[b]).wait()

        @pl.when(g >= 2)
        def _():
            base2 = wid * EW + (g - 2) * CEB
            pltpu.make_async_copy(ext, ext_h.at[:, pl.ds(base2, CEB)], wses[b]).wait()
            pltpu.make_async_copy(exr, den_sp.at[dstb.at[g]], wsds[b]).wait()

        def edge(e, ecarry):
            sums = jnp.zeros((16,), jnp.float32)
            for h in range(H1):
                p = None
                for j in range(4):
                    kv = kr[e, pl.ds(h * D1 + j * 16, 16)]
                    qv = qr[e, pl.ds(h * D1 + j * 16, 16)]
                    t = kv * qv
                    p = t if p is None else p + t
                sums = jnp.where(iota == h, _sum16(p), sums)
            ex = jnp.exp(sums)
            exr[e, :] = ex
            plsc.store_scatter(ext, [iota, jnp.full((16,), e, jnp.int32)], ex)
            return ecarry
        lax.fori_loop(0, CEB, edge, 0, unroll=8)
        base = wid * EW + g * CEB
        pltpu.async_copy(ext, ext_h.at[:, pl.ds(base, CEB)], wses[b])
        pltpu.async_copy(exr, den_sp.at[dstb.at[g]], wsds[b], add=True)

    def chunk(g, carry):
        @pl.when(g % 2 == 0)
        def _():
            half(0, g)

        @pl.when(g % 2 == 1)
        def _():
            half(1, g)
        return carry
    lax.fori_loop(0, NCHB, chunk, 0)

    pltpu.make_async_copy(k_h.at[srcb.at[0]], kr0, semk0).wait()
    pltpu.make_async_copy(q_h.at[dstb.at[0]], qr0, semq0).wait()
    for b in range(2):
        base = wid * EW + (NCHB - 2 + b) * CEB
        pltpu.make_async_copy(exts[b], ext_h.at[:, pl.ds(base, CEB)], wses[b]).wait()
        pltpu.make_async_copy(exrs[b], den_sp.at[dstb.at[0]], wsds[b]).wait()
    plsc.subcore_barrier()
    pltpu.sync_copy(den_sp.at[pl.ds(s * RPS, RPS)],
                    den_h.at[c, pl.ds(s * RPS, RPS)])


_sc_params = pltpu.CompilerParams(needs_layout_passes=False, use_tc_tiling_on_sc=False)

_kb = pl.kernel(
    _kb_body,
    out_type=[jax.ShapeDtypeStruct((16, E), jnp.float32),
              jax.ShapeDtypeStruct((NC, NP, 16), jnp.float32)],
    mesh=_mesh,
    compiler_params=_sc_params,
    scratch_types=[pltpu.VMEM((NCHB, CEB), jnp.int32), pltpu.VMEM((NCHB, CEB), jnp.int32),
                   pltpu.VMEM((CEB, HID), jnp.float32), pltpu.VMEM((CEB, HID), jnp.float32),
                   pltpu.VMEM((CEB, HID), jnp.float32), pltpu.VMEM((CEB, HID), jnp.float32),
                   pltpu.VMEM((CEB, 16), jnp.float32), pltpu.VMEM((CEB, 16), jnp.float32),
                   pltpu.VMEM((16, CEB), jnp.float32), pltpu.VMEM((16, CEB), jnp.float32),
                   pltpu.VMEM((H1 * 16,), jnp.float32),
                   pltpu.VMEM((RPS, 16), jnp.float32),
                   pltpu.VMEM_SHARED((NP, 16), jnp.float32)] +
                  [pltpu.SemaphoreType.DMA] * 8,
    name="hgt_logits",
)


CEA = 400
NCHA = EW // CEA

def _ke_body(v_h, src3_h, dst3_h, ext_h, zeros_h, out_h,
             sidx0, sidx1, dstb, vr0, vr1, msg, exc, acc_sp,
             semv0, semv1, wsm):
    c = lax.axis_index("c")
    s = lax.axis_index("s")
    wid = s * NC + c
    vrs = (vr0, vr1)
    sidxs = (sidx0, sidx1)
    semvs = (semv0, semv1)

    def icopy(t, carry):
        pltpu.sync_copy(dst3_h.at[wid, pl.ds(t * 5, 5)], dstb.at[pl.ds(t * 5, 5)])
        return carry
    lax.fori_loop(0, NCHA // 5, icopy, 0)

    def load_adj_idx(nb, g, fc):
        pltpu.sync_copy(src3_h.at[wid, g], sidxs[nb])
        off = fc * N

        def adj(k, carry):
            sl = pl.ds(k * 16, 16)
            sidxs[nb][sl] = sidxs[nb][sl] + off
            return carry
        lax.fori_loop(0, CEA // 16, adj, 0)

    def fcpass(fc, fcarry):
        pltpu.sync_copy(zeros_h.at[pl.ds(s * RPS, RPS)],
                        acc_sp.at[pl.ds(s * RPS, RPS)])
        plsc.subcore_barrier()

        load_adj_idx(0, 0, fc)
        pltpu.async_copy(v_h.at[sidx0], vr0, semv0)

        def dma_part(b, g):
            nb = 1 - b
            gp = jnp.minimum(g + 1, NCHA - 1)
            load_adj_idx(nb, gp, fc)
            pltpu.async_copy(v_h.at[sidxs[nb]], vrs[nb], semvs[nb])
            base = wid * EW + g * CEA
            pltpu.sync_copy(ext_h.at[fc, pl.ds(base, CEA)], exc)
            pltpu.make_async_copy(v_h.at[sidxs[b]], vrs[b], semvs[b]).wait()

        def calc_part(b, g):
            vr = vrs[b]

            def sub(i, icarry):
                wv = exc[pl.ds(i * 16, 16)]
                for el in range(16):
                    e = i * 16 + el
                    w = _lane_bcast(wv, el)
                    for j in range(FW // 16):
                        msg[e, pl.ds(j * 16, 16)] = vr[e, pl.ds(j * 16, 16)] * w
                return icarry
            lax.fori_loop(0, CEA // 16, sub, 0)

        def chunk(g, carry):
            gc = jnp.minimum(g, NCHA - 1)

            @pl.when(g < NCHA)
            def _():
                @pl.when(g % 2 == 0)
                def _():
                    dma_part(0, g)

                @pl.when(g % 2 == 1)
                def _():
                    dma_part(1, g)

            @pl.when(g >= 1)
            def _():
                pltpu.make_async_copy(msg, acc_sp.at[dstb.at[gc]], wsm).wait()

            @pl.when(g < NCHA)
            def _():
                @pl.when(g % 2 == 0)
                def _():
                    calc_part(0, g)

                @pl.when(g % 2 == 1)
                def _():
                    calc_part(1, g)
                pltpu.async_copy(msg, acc_sp.at[dstb.at[gc]], wsm, add=True)
            return carry
        lax.fori_loop(0, NCHA + 1, chunk, 0)

        pltpu.make_async_copy(v_h.at[sidx1], vr1, semv1).wait()
        plsc.subcore_barrier()
        pltpu.sync_copy(acc_sp.at[pl.ds(s * RPS, RPS)],
                        out_h.at[fc, c, pl.ds(s * RPS, RPS)])
        return fcarry
    lax.fori_loop(0, FCH, fcpass, 0)


_ke = pl.kernel(
    _ke_body,
    out_type=jax.ShapeDtypeStruct((FCH, NC, NP, FW), jnp.float32),
    mesh=_mesh,
    compiler_params=_sc_params,
    scratch_types=[pltpu.VMEM((CEA,), jnp.int32), pltpu.VMEM((CEA,), jnp.int32),
                   pltpu.VMEM((NCHA, CEA), jnp.int32),
                   pltpu.VMEM((CEA, FW), jnp.float32), pltpu.VMEM((CEA, FW), jnp.float32),
                   pltpu.VMEM((CEA, FW), jnp.float32),
                   pltpu.VMEM((CEA,), jnp.float32),
                   pltpu.VMEM_SHARED((NP, FW), jnp.float32)] +
                  [pltpu.SemaphoreType.DMA] * 3,
    name="hgt_agg1",
)


def _kf_body(op_ref, dp_ref, wa_ref, w2_ref, k2_ref, q2_ref, v2_ref):
    dp = dp_ref[...]
    rden = 1.0 / (dp[0] + dp[1] + 1e-16)
    cols = []
    for c in range(FCH):
        part = op_ref[c, 0] + op_ref[c, 1]
        cols.append(part * rden[:, c][:, None])
    agg = jnp.concatenate(cols, axis=1)
    h = jnp.dot(jax.nn.gelu(agg), wa_ref[...], preferred_element_type=jnp.float32)
    kqv = jnp.dot(h, w2_ref[...], preferred_element_type=jnp.float32)
    k2_ref[...] = kqv[:, 0:2]
    q2_ref[...] = kqv[:, 2:4]
    v2_ref[...] = kqv[:, 4:6]


def _kf(outp, denp, wa1, w2cat):
    return pl.pallas_call(
        _kf_body,
        grid=(N // RB,),
        in_specs=[pl.BlockSpec((FCH, NC, RB, FW), lambda i: (0, 0, i, 0)),
                 pl.BlockSpec((NC, RB, 16), lambda i: (0, i, 0)),
                  pl.BlockSpec((HID, HID), lambda i: (0, 0)),
                  pl.BlockSpec((HID, 6), lambda i: (0, 0))],
        out_specs=[pl.BlockSpec((RB, 2), lambda i: (i, 0))] * 3,
        out_shape=[jax.ShapeDtypeStruct((N, 2), jnp.float32)] * 3,
    )(outp, denp, wa1, w2cat)


def _kg_body(k2_h, q2_h, v2_h, src_h, dst3_h, acc_h,
             k2t, q2t, v2t, srcb, dst2, msg0, msg1, zbuf, acc_sp, wsm0, wsm1):
    c = lax.axis_index("c")
    s = lax.axis_index("s")
    wid = s * NC + c
    iota = lax.iota(jnp.int32, 16)
    z16 = jnp.zeros((16,), jnp.int32)
    o16 = jnp.ones((16,), jnp.int32)
    msgs = (msg0, msg1)
    wsms = (wsm0, wsm1)

    pltpu.sync_copy(k2_h, k2t)
    pltpu.sync_copy(q2_h, q2t)
    pltpu.sync_copy(v2_h, v2t)
    pltpu.sync_copy(src_h.at[pl.ds(wid * EW, EW)], srcb)

    def icopy(t, carry):
        pltpu.sync_copy(dst3_h.at[wid, pl.ds(t * 5, 5)], dst2.at[pl.ds(t * 5, 5)])
        return carry
    lax.fori_loop(0, NCH // 5, icopy, 0)

    def zm(i, carry):
        msg0[i, :] = jnp.zeros((16,), jnp.float32)
        msg1[i, :] = jnp.zeros((16,), jnp.float32)
        return carry
    lax.fori_loop(0, CE, zm, 0)

    def zrow(i, carry):
        zbuf[i, :] = jnp.zeros((16,), jnp.float32)
        return carry
    lax.fori_loop(0, RPS, zrow, 0)
    pltpu.sync_copy(zbuf, acc_sp.at[pl.ds(s * RPS, RPS)])
    plsc.subcore_barrier()

    def half(b, g):
        msg = msgs[b]

        @pl.when(g >= 2)
        def _():
            pltpu.make_async_copy(msg, acc_sp.at[dst2.at[g]], wsms[b]).wait()

        def sub(i, icarry):
            sv = srcb[pl.ds(g * CE + i * 16, 16)]
            dv = dst2[g, pl.ds(i * 16, 16)]
            kx = plsc.load_gather(k2t, [z16, sv])
            ky = plsc.load_gather(k2t, [o16, sv])
            qx = plsc.load_gather(q2t, [z16, dv])
            qy = plsc.load_gather(q2t, [o16, dv])
            ex2 = jnp.exp(kx * qx + ky * qy)
            vx = plsc.load_gather(v2t, [z16, sv])
            vy = plsc.load_gather(v2t, [o16, sv])
            rows = i * 16 + iota
            plsc.store_scatter(msg, [rows, z16], ex2)
            plsc.store_scatter(msg, [rows, o16], vx * ex2)
            plsc.store_scatter(msg, [rows, o16 + 1], vy * ex2)
            return icarry
        lax.fori_loop(0, CE // 16, sub, 0)
        pltpu.async_copy(msg, acc_sp.at[dst2.at[g]], wsms[b], add=True)

    def chunk(g, carry):
        @pl.when(g % 2 == 0)
        def _():
            half(0, g)

        @pl.when(g % 2 == 1)
        def _():
            half(1, g)
        return carry
    lax.fori_loop(0, NCH, chunk, 0)
    for b in range(2):
        pltpu.make_async_copy(msgs[b], acc_sp.at[dst2.at[0]], wsms[b]).wait()
    plsc.subcore_barrier()
    pltpu.sync_copy(acc_sp.at[pl.ds(s * RPS, RPS)],
                    acc_h.at[c, pl.ds(s * RPS, RPS)])


_kg = pl.kernel(
    _kg_body,
    out_type=jax.ShapeDtypeStruct((NC, NP, 16), jnp.float32),
    mesh=_mesh,
    compiler_params=_sc_params,
    scratch_types=[pltpu.VMEM((2, N), jnp.float32), pltpu.VMEM((2, N), jnp.float32),
                   pltpu.VMEM((2, N), jnp.float32),
                   pltpu.VMEM((EW,), jnp.int32), pltpu.VMEM((NCH, CE), jnp.int32),
                   pltpu.VMEM((CE, 16), jnp.float32), pltpu.VMEM((CE, 16), jnp.float32),
                   pltpu.VMEM((RPS, 16), jnp.float32),
                   pltpu.VMEM_SHARED((NP, 16), jnp.float32),
                   pltpu.SemaphoreType.DMA, pltpu.SemaphoreType.DMA],
)


def _kh_body(a_ref, wa2_ref, o_ref):
    a = a_ref[0] + a_ref[1]
    den = a[:, 0:1]
    num = a[:, 1:3]
    o_ref[...] = jnp.dot(jax.nn.gelu(num / (den + 1e-16)), wa2_ref[...],
                         preferred_element_type=jnp.float32)


def _kh(acc2, wa2):
    return pl.pallas_call(
        _kh_body,
        grid=(N // RB,),
        in_specs=[pl.BlockSpec((NC, RB, 16), lambda i: (0, i, 0)),
                  pl.BlockSpec((2, 2), lambda i: (0, 0))],
        out_specs=pl.BlockSpec((RB, 2), lambda i: (i, 0)),
        out_shape=jax.ShapeDtypeStruct((N, 2), jnp.float32),
    )(acc2, wa2)


def kernel(x, edge_index, Wk1, Wq1, Wv1, a_rel1, m_rel1, pri1, Wa1,
           Wk2, Wq2, Wv2, a_rel2, m_rel2, pri2, Wa2):
    ei = edge_index.astype(jnp.int32)
    src, dst = ei[0], ei[1]

    scale1 = pri1 / jnp.sqrt(jnp.float32(D1))
    wk1e = jnp.einsum("ihd,hde->ihe", Wk1.reshape(D_IN, H1, D1), a_rel1)
    wq1s = Wq1.reshape(D_IN, H1, D1) * scale1[None, :, None]
    wv1e = jnp.einsum("ihd,hde->ihe", Wv1.reshape(D_IN, H1, D1), m_rel1)
    wcat = jnp.concatenate([wk1e.reshape(D_IN, HID), wq1s.reshape(D_IN, HID),
                            wv1e.reshape(D_IN, HID)], axis=1)

    proj_out = _proj(x, wcat)
    k1, q1, vs = proj_out[0], proj_out[1], proj_out[2:]
    src3b = src.reshape(NW, NCHB, CEB)
    dst3b = dst.reshape(NW, NCHB, CEB)
    src3e = src.reshape(NW, NCHA, CEA)
    dst3e = dst.reshape(NW, NCHA, CEA)
    ext, denp = _kb(k1, q1, src3b, dst3b)
    zeros_big = jnp.zeros((NP, FW), jnp.float32)
    vcat = jnp.concatenate(vs, axis=0)
    outp = _ke(vcat, src3e, dst3e, ext, zeros_big)

    d2 = OUT
    w2k = Wk2 @ a_rel2[0]
    w2q = Wq2 * (pri2[0] / jnp.sqrt(jnp.float32(d2)))
    w2v = Wv2 @ m_rel2[0]
    w2cat = jnp.concatenate([w2k, w2q, w2v], axis=1)

    k2, q2, v2n = _kf(outp, denp, Wa1, w2cat)
    dst3g = dst.reshape(NW, NCH, CE)
    acc2 = _kg(k2.T, q2.T, v2n.T, src, dst3g)
    return _kh(acc2, Wa2)


# speedup vs baseline: 1.0578x; 1.0578x over previous
import functools

import jax
import jax.numpy as jnp
from jax import lax
from jax.experimental import pallas as pl
from jax.experimental.pallas import tpu as pltpu
from jax.experimental.pallas import tpu_sc as plsc

N = 10000
E = 320000
D_IN = 128
HID = 512
H1 = 8
D1 = 64
OUT = 2

NC = 2
NS = 16
NW = NC * NS
EW = E // NW
CE = 80
NCH = EW // CE
CEB = 40
NCHB = EW // CEB
NP = 10112
RPS = NP // NS
FCH = 8
FW = HID // FCH
RB = 1000

_mesh = plsc.VectorSubcoreMesh(core_axis_name="c", subcore_axis_name="s")

_GDN = lax.GatherDimensionNumbers(offset_dims=(), collapsed_slice_dims=(0,),
                                  start_index_map=(0,))


def _lane_bcast(v, lane):
    idx = jnp.full((16, 1), lane, jnp.int32)
    return lax.gather(v, idx, _GDN, (1,),
                      mode=lax.GatherScatterMode.PROMISE_IN_BOUNDS)


def _lane_rot(v, k):
    idx = ((lax.iota(jnp.int32, 16) + k) & 15).reshape(16, 1)
    return lax.gather(v, idx, _GDN, (1,),
                      mode=lax.GatherScatterMode.PROMISE_IN_BOUNDS)


def _sum16(v):
    for k in (8, 4, 2, 1):
        v = v + _lane_rot(v, k)
    return v


def _proj_body(x_ref, w_ref, k_ref, q_ref, *v_refs):
    y = jnp.dot(x_ref[...], w_ref[...], preferred_element_type=jnp.float32)
    k_ref[...] = y[:, :HID]
    q_ref[...] = y[:, HID:2 * HID]
    for c, vr in enumerate(v_refs):
        vr[...] = y[:, 2 * HID + c * FW:2 * HID + (c + 1) * FW]


def _proj(x, wcat):
    return pl.pallas_call(
        _proj_body,
        grid=(N // RB,),
        in_specs=[pl.BlockSpec((RB, D_IN), lambda i: (i, 0)),
                  pl.BlockSpec((D_IN, 3 * HID), lambda i: (0, 0))],
        out_specs=[pl.BlockSpec((RB, HID), lambda i: (i, 0)),
                   pl.BlockSpec((RB, HID), lambda i: (i, 0))] +
                  [pl.BlockSpec((RB, FW), lambda i: (i, 0))] * FCH,
        out_shape=[jax.ShapeDtypeStruct((N, HID), jnp.float32)] * 2 +
                  [jax.ShapeDtypeStruct((N, FW), jnp.float32)] * FCH,
    )(x, wcat)


def _kb_body(k_h, q_h, src3_h, dst3_h, ext_h, den_h,
             srcb, dstb, kr0, kr1, qr0, qr1, exr0, exr1, ext0, ext1, cs, zbuf,
             den_sp, semk0, semk1, semq0, semq1, wse0, wse1, wsd0, wsd1):
    c = lax.axis_index("c")
    s = lax.axis_index("s")
    wid = s * NC + c
    iota = lax.iota(jnp.int32, 16)
    gidx = jnp.minimum(iota, H1 - 1) * 16 + 15
    krs = (kr0, kr1)
    qrs = (qr0, qr1)
    exrs = (exr0, exr1)
    exts = (ext0, ext1)
    semks = (semk0, semk1)
    semqs = (semq0, semq1)
    wses = (wse0, wse1)
    wsds = (wsd0, wsd1)

    def zrow(i, carry):
        zbuf[i, :] = jnp.zeros((16,), jnp.float32)
        return carry
    lax.fori_loop(0, RPS, zrow, 0)
    pltpu.sync_copy(zbuf, den_sp.at[pl.ds(s * RPS, RPS)])
    pltpu.sync_copy(src3_h.at[wid], srcb)
    pltpu.sync_copy(dst3_h.at[wid], dstb)
    plsc.subcore_barrier()

    pltpu.async_copy(k_h.at[srcb.at[0]], kr0, semk0)
    pltpu.async_copy(q_h.at[dstb.at[0]], qr0, semq0)

    def half(b, g):
        kr, qr, exr, ext = krs[b], qrs[b], exrs[b], exts[b]
        nb = 1 - b
        gp = jnp.minimum(g + 1, NCHB - 1)
        pltpu.async_copy(k_h.at[srcb.at[gp]], krs[nb], semks[nb])
        pltpu.async_copy(q_h.at[dstb.at[gp]], qrs[nb], semqs[nb])
        pltpu.make_async_copy(k_h.at[srcb.at[g]], kr, semks[b]).wait()
        pltpu.make_async_copy(q_h.at[dstb.at[g]], qr, semqs[b]).wait()

        @pl.when(g >= 2)
        def _():
            base2 = wid * EW + (g - 2) * CEB
            pltpu.make_async_copy(ext, ext_h.at[:, pl.ds(base2, CEB)], wses[b]).wait()
            pltpu.make_async_copy(exr, den_sp.at[dstb.at[g]], wsds[b]).wait()

        def edge(e, ecarry):
            sums = jnp.zeros((16,), jnp.float32)
            for h in range(H1):
                p = None
                for j in range(4):
                    kv = kr[e, pl.ds(h * D1 + j * 16, 16)]
                    qv = qr[e, pl.ds(h * D1 + j * 16, 16)]
                    t = kv * qv
                    p = t if p is None else p + t
                sums = jnp.where(iota == h, _sum16(p), sums)
            ex = jnp.exp(sums)
            exr[e, :] = ex
            plsc.store_scatter(ext, [iota, jnp.full((16,), e, jnp.int32)], ex)
            return ecarry
        lax.fori_loop(0, CEB, edge, 0, unroll=4)
        base = wid * EW + g * CEB
        pltpu.async_copy(ext, ext_h.at[:, pl.ds(base, CEB)], wses[b])
        pltpu.async_copy(exr, den_sp.at[dstb.at[g]], wsds[b], add=True)

    def chunk(g, carry):
        @pl.when(g % 2 == 0)
        def _():
            half(0, g)

        @pl.when(g % 2 == 1)
        def _():
            half(1, g)
        return carry
    lax.fori_loop(0, NCHB, chunk, 0)

    pltpu.make_async_copy(k_h.at[srcb.at[0]], kr0, semk0).wait()
    pltpu.make_async_copy(q_h.at[dstb.at[0]], qr0, semq0).wait()
    for b in range(2):
        base = wid * EW + (NCHB - 2 + b) * CEB
        pltpu.make_async_copy(exts[b], ext_h.at[:, pl.ds(base, CEB)], wses[b]).wait()
        pltpu.make_async_copy(exrs[b], den_sp.at[dstb.at[0]], wsds[b]).wait()
    plsc.subcore_barrier()
    pltpu.sync_copy(den_sp.at[pl.ds(s * RPS, RPS)],
                    den_h.at[c, pl.ds(s * RPS, RPS)])


_sc_params = pltpu.CompilerParams(needs_layout_passes=False, use_tc_tiling_on_sc=False)

_kb = pl.kernel(
    _kb_body,
    out_type=[jax.ShapeDtypeStruct((16, E), jnp.float32),
              jax.ShapeDtypeStruct((NC, NP, 16), jnp.float32)],
    mesh=_mesh,
    compiler_params=_sc_params,
    scratch_types=[pltpu.VMEM((NCHB, CEB), jnp.int32), pltpu.VMEM((NCHB, CEB), jnp.int32),
                   pltpu.VMEM((CEB, HID), jnp.float32), pltpu.VMEM((CEB, HID), jnp.float32),
                   pltpu.VMEM((CEB, HID), jnp.float32), pltpu.VMEM((CEB, HID), jnp.float32),
                   pltpu.VMEM((CEB, 16), jnp.float32), pltpu.VMEM((CEB, 16), jnp.float32),
                   pltpu.VMEM((16, CEB), jnp.float32), pltpu.VMEM((16, CEB), jnp.float32),
                   pltpu.VMEM((H1 * 16,), jnp.float32),
                   pltpu.VMEM((RPS, 16), jnp.float32),
                   pltpu.VMEM_SHARED((NP, 16), jnp.float32)] +
                  [pltpu.SemaphoreType.DMA] * 8,
    name="hgt_logits",
)


CEA = 400
NCHA = EW // CEA

def _ke_body(v_h, src3_h, dst3_h, ext_h, zeros_h, out_h,
             sidx0, sidx1, dstb, vr0, vr1, msg, exc, acc_sp,
             semv0, semv1, wsm):
    c = lax.axis_index("c")
    s = lax.axis_index("s")
    wid = s * NC + c
    vrs = (vr0, vr1)
    sidxs = (sidx0, sidx1)
    semvs = (semv0, semv1)

    def icopy(t, carry):
        pltpu.sync_copy(dst3_h.at[wid, pl.ds(t * 5, 5)], dstb.at[pl.ds(t * 5, 5)])
        return carry
    lax.fori_loop(0, NCHA // 5, icopy, 0)

    def load_adj_idx(nb, g, fc):
        pltpu.sync_copy(src3_h.at[wid, g], sidxs[nb])
        off = fc * N

        def adj(k, carry):
            sl = pl.ds(k * 16, 16)
            sidxs[nb][sl] = sidxs[nb][sl] + off
            return carry
        lax.fori_loop(0, CEA // 16, adj, 0)

    def fcpass(fc, fcarry):
        pltpu.sync_copy(zeros_h.at[pl.ds(s * RPS, RPS)],
                        acc_sp.at[pl.ds(s * RPS, RPS)])
        plsc.subcore_barrier()

        load_adj_idx(0, 0, fc)
        pltpu.async_copy(v_h.at[sidx0], vr0, semv0)

        def dma_part(b, g):
            nb = 1 - b
            gp = jnp.minimum(g + 1, NCHA - 1)
            load_adj_idx(nb, gp, fc)
            pltpu.async_copy(v_h.at[sidxs[nb]], vrs[nb], semvs[nb])
            base = wid * EW + g * CEA
            pltpu.sync_copy(ext_h.at[fc, pl.ds(base, CEA)], exc)
            pltpu.make_async_copy(v_h.at[sidxs[b]], vrs[b], semvs[b]).wait()

        def calc_part(b, g):
            vr = vrs[b]

            def sub(i, icarry):
                wv = exc[pl.ds(i * 16, 16)]
                for el in range(16):
                    e = i * 16 + el
                    w = _lane_bcast(wv, el)
                    for j in range(FW // 16):
                        msg[e, pl.ds(j * 16, 16)] = vr[e, pl.ds(j * 16, 16)] * w
                return icarry
            lax.fori_loop(0, CEA // 16, sub, 0)

        def chunk(g, carry):
            gc = jnp.minimum(g, NCHA - 1)

            @pl.when(g < NCHA)
            def _():
                @pl.when(g % 2 == 0)
                def _():
                    dma_part(0, g)

                @pl.when(g % 2 == 1)
                def _():
                    dma_part(1, g)

            @pl.when(g >= 1)
            def _():
                pltpu.make_async_copy(msg, acc_sp.at[dstb.at[gc]], wsm).wait()

            @pl.when(g < NCHA)
            def _():
                @pl.when(g % 2 == 0)
                def _():
                    calc_part(0, g)

                @pl.when(g % 2 == 1)
                def _():
                    calc_part(1, g)
                pltpu.async_copy(msg, acc_sp.at[dstb.at[gc]], wsm, add=True)
            return carry
        lax.fori_loop(0, NCHA + 1, chunk, 0)

        pltpu.make_async_copy(v_h.at[sidx1], vr1, semv1).wait()
        plsc.subcore_barrier()
        pltpu.sync_copy(acc_sp.at[pl.ds(s * RPS, RPS)],
                        out_h.at[fc, c, pl.ds(s * RPS, RPS)])
        return fcarry
    lax.fori_loop(0, FCH, fcpass, 0)


_ke = pl.kernel(
    _ke_body,
    out_type=jax.ShapeDtypeStruct((FCH, NC, NP, FW), jnp.float32),
    mesh=_mesh,
    compiler_params=_sc_params,
    scratch_types=[pltpu.VMEM((CEA,), jnp.int32), pltpu.VMEM((CEA,), jnp.int32),
                   pltpu.VMEM((NCHA, CEA), jnp.int32),
                   pltpu.VMEM((CEA, FW), jnp.float32), pltpu.VMEM((CEA, FW), jnp.float32),
                   pltpu.VMEM((CEA, FW), jnp.float32),
                   pltpu.VMEM((CEA,), jnp.float32),
                   pltpu.VMEM_SHARED((NP, FW), jnp.float32)] +
                  [pltpu.SemaphoreType.DMA] * 3,
    name="hgt_agg1",
)


def _kf_body(op_ref, dp_ref, wa_ref, w2_ref, k2_ref, q2_ref, v2_ref):
    dp = dp_ref[...]
    rden = 1.0 / (dp[0] + dp[1] + 1e-16)
    cols = []
    for c in range(FCH):
        part = op_ref[c, 0] + op_ref[c, 1]
        cols.append(part * rden[:, c][:, None])
    agg = jnp.concatenate(cols, axis=1)
    h = jnp.dot(jax.nn.gelu(agg), wa_ref[...], preferred_element_type=jnp.float32)
    kqv = jnp.dot(h, w2_ref[...], preferred_element_type=jnp.float32)
    k2_ref[...] = kqv[:, 0:2]
    q2_ref[...] = kqv[:, 2:4]
    v2_ref[...] = kqv[:, 4:6]


def _kf(outp, denp, wa1, w2cat):
    return pl.pallas_call(
        _kf_body,
        grid=(N // RB,),
        in_specs=[pl.BlockSpec((FCH, NC, RB, FW), lambda i: (0, 0, i, 0)),
                 pl.BlockSpec((NC, RB, 16), lambda i: (0, i, 0)),
                  pl.BlockSpec((HID, HID), lambda i: (0, 0)),
                  pl.BlockSpec((HID, 6), lambda i: (0, 0))],
        out_specs=[pl.BlockSpec((RB, 2), lambda i: (i, 0))] * 3,
        out_shape=[jax.ShapeDtypeStruct((N, 2), jnp.float32)] * 3,
    )(outp, denp, wa1, w2cat)


def _kg_body(k2_h, q2_h, v2_h, src_h, dst3_h, acc_h,
             k2t, q2t, v2t, srcb, dst2, msg0, msg1, zbuf, acc_sp, wsm0, wsm1):
    c = lax.axis_index("c")
    s = lax.axis_index("s")
    wid = s * NC + c
    iota = lax.iota(jnp.int32, 16)
    z16 = jnp.zeros((16,), jnp.int32)
    o16 = jnp.ones((16,), jnp.int32)
    msgs = (msg0, msg1)
    wsms = (wsm0, wsm1)

    pltpu.sync_copy(k2_h, k2t)
    pltpu.sync_copy(q2_h, q2t)
    pltpu.sync_copy(v2_h, v2t)
    pltpu.sync_copy(src_h.at[pl.ds(wid * EW, EW)], srcb)

    def icopy(t, carry):
        pltpu.sync_copy(dst3_h.at[wid, pl.ds(t * 5, 5)], dst2.at[pl.ds(t * 5, 5)])
        return carry
    lax.fori_loop(0, NCH // 5, icopy, 0)

    def zm(i, carry):
        msg0[i, :] = jnp.zeros((16,), jnp.float32)
        msg1[i, :] = jnp.zeros((16,), jnp.float32)
        return carry
    lax.fori_loop(0, CE, zm, 0)

    def zrow(i, carry):
        zbuf[i, :] = jnp.zeros((16,), jnp.float32)
        return carry
    lax.fori_loop(0, RPS, zrow, 0)
    pltpu.sync_copy(zbuf, acc_sp.at[pl.ds(s * RPS, RPS)])
    plsc.subcore_barrier()

    def half(b, g):
        msg = msgs[b]

        @pl.when(g >= 2)
        def _():
            pltpu.make_async_copy(msg, acc_sp.at[dst2.at[g]], wsms[b]).wait()

        def sub(i, icarry):
            sv = srcb[pl.ds(g * CE + i * 16, 16)]
            dv = dst2[g, pl.ds(i * 16, 16)]
            kx = plsc.load_gather(k2t, [z16, sv])
            ky = plsc.load_gather(k2t, [o16, sv])
            qx = plsc.load_gather(q2t, [z16, dv])
            qy = plsc.load_gather(q2t, [o16, dv])
            ex2 = jnp.exp(kx * qx + ky * qy)
            vx = plsc.load_gather(v2t, [z16, sv])
            vy = plsc.load_gather(v2t, [o16, sv])
            rows = i * 16 + iota
            plsc.store_scatter(msg, [rows, z16], ex2)
            plsc.store_scatter(msg, [rows, o16], vx * ex2)
            plsc.store_scatter(msg, [rows, o16 + 1], vy * ex2)
            return icarry
        lax.fori_loop(0, CE // 16, sub, 0)
        pltpu.async_copy(msg, acc_sp.at[dst2.at[g]], wsms[b], add=True)

    def chunk(g, carry):
        @pl.when(g % 2 == 0)
        def _():
            half(0, g)

        @pl.when(g % 2 == 1)
        def _():
            half(1, g)
        return carry
    lax.fori_loop(0, NCH, chunk, 0)
    for b in range(2):
        pltpu.make_async_copy(msgs[b], acc_sp.at[dst2.at[0]], wsms[b]).wait()
    plsc.subcore_barrier()
    pltpu.sync_copy(acc_sp.at[pl.ds(s * RPS, RPS)],
                    acc_h.at[c, pl.ds(s * RPS, RPS)])


_kg = pl.kernel(
    _kg_body,
    out_type=jax.ShapeDtypeStruct((NC, NP, 16), jnp.float32),
    mesh=_mesh,
    compiler_params=_sc_params,
    scratch_types=[pltpu.VMEM((2, N), jnp.float32), pltpu.VMEM((2, N), jnp.float32),
                   pltpu.VMEM((2, N), jnp.float32),
                   pltpu.VMEM((EW,), jnp.int32), pltpu.VMEM((NCH, CE), jnp.int32),
                   pltpu.VMEM((CE, 16), jnp.float32), pltpu.VMEM((CE, 16), jnp.float32),
                   pltpu.VMEM((RPS, 16), jnp.float32),
                   pltpu.VMEM_SHARED((NP, 16), jnp.float32),
                   pltpu.SemaphoreType.DMA, pltpu.SemaphoreType.DMA],
)


def _kh_body(a_ref, wa2_ref, o_ref):
    a = a_ref[0] + a_ref[1]
    den = a[:, 0:1]
    num = a[:, 1:3]
    o_ref[...] = jnp.dot(jax.nn.gelu(num / (den + 1e-16)), wa2_ref[...],
                         preferred_element_type=jnp.float32)


def _kh(acc2, wa2):
    return pl.pallas_call(
        _kh_body,
        grid=(N // RB,),
        in_specs=[pl.BlockSpec((NC, RB, 16), lambda i: (0, i, 0)),
                  pl.BlockSpec((2, 2), lambda i: (0, 0))],
        out_specs=pl.BlockSpec((RB, 2), lambda i: (i, 0)),
        out_shape=jax.ShapeDtypeStruct((N, 2), jnp.float32),
    )(acc2, wa2)


def kernel(x, edge_index, Wk1, Wq1, Wv1, a_rel1, m_rel1, pri1, Wa1,
           Wk2, Wq2, Wv2, a_rel2, m_rel2, pri2, Wa2):
    ei = edge_index.astype(jnp.int32)
    src, dst = ei[0], ei[1]

    scale1 = pri1 / jnp.sqrt(jnp.float32(D1))
    wk1e = jnp.einsum("ihd,hde->ihe", Wk1.reshape(D_IN, H1, D1), a_rel1)
    wq1s = Wq1.reshape(D_IN, H1, D1) * scale1[None, :, None]
    wv1e = jnp.einsum("ihd,hde->ihe", Wv1.reshape(D_IN, H1, D1), m_rel1)
    wcat = jnp.concatenate([wk1e.reshape(D_IN, HID), wq1s.reshape(D_IN, HID),
                            wv1e.reshape(D_IN, HID)], axis=1)

    proj_out = _proj(x, wcat)
    k1, q1, vs = proj_out[0], proj_out[1], proj_out[2:]
    src3b = src.reshape(NW, NCHB, CEB)
    dst3b = dst.reshape(NW, NCHB, CEB)
    src3e = src.reshape(NW, NCHA, CEA)
    dst3e = dst.reshape(NW, NCHA, CEA)
    ext, denp = _kb(k1, q1, src3b, dst3b)
    zeros_big = jnp.zeros((NP, FW), jnp.float32)
    vcat = jnp.concatenate(vs, axis=0)
    outp = _ke(vcat, src3e, dst3e, ext, zeros_big)

    d2 = OUT
    w2k = Wk2 @ a_rel2[0]
    w2q = Wq2 * (pri2[0] / jnp.sqrt(jnp.float32(d2)))
    w2v = Wv2 @ m_rel2[0]
    w2cat = jnp.concatenate([w2k, w2q, w2v], axis=1)

    k2, q2, v2n = _kf(outp, denp, Wa1, w2cat)
    dst3g = dst.reshape(NW, NCH, CE)
    acc2 = _kg(k2.T, q2.T, v2n.T, src, dst3g)
    return _kh(acc2, Wa2)


# speedup vs baseline: 1.0679x; 1.0095x over previous
import functools

import jax
import jax.numpy as jnp
from jax import lax
from jax.experimental import pallas as pl
from jax.experimental.pallas import tpu as pltpu
from jax.experimental.pallas import tpu_sc as plsc

N = 10000
E = 320000
D_IN = 128
HID = 512
H1 = 8
D1 = 64
OUT = 2

NC = 2
NS = 16
NW = NC * NS
EW = E // NW
CE = 80
NCH = EW // CE
CEB = 40
NCHB = EW // CEB
NP = 10112
RPS = NP // NS
FCH = 8
FW = HID // FCH
RB = 1000

_mesh = plsc.VectorSubcoreMesh(core_axis_name="c", subcore_axis_name="s")

_GDN = lax.GatherDimensionNumbers(offset_dims=(), collapsed_slice_dims=(0,),
                                  start_index_map=(0,))


def _lane_bcast(v, lane):
    idx = jnp.full((16, 1), lane, jnp.int32)
    return lax.gather(v, idx, _GDN, (1,),
                      mode=lax.GatherScatterMode.PROMISE_IN_BOUNDS)


def _lane_rot(v, k):
    idx = ((lax.iota(jnp.int32, 16) + k) & 15).reshape(16, 1)
    return lax.gather(v, idx, _GDN, (1,),
                      mode=lax.GatherScatterMode.PROMISE_IN_BOUNDS)


def _sum16(v):
    for k in (8, 4, 2, 1):
        v = v + _lane_rot(v, k)
    return v


def _proj_body(x_ref, w_ref, k_ref, q_ref, v_ref):
    y = jnp.dot(x_ref[...], w_ref[...], preferred_element_type=jnp.float32)
    k_ref[...] = y[:, :HID]
    q_ref[...] = y[:, HID:2 * HID]
    v = y[:, 2 * HID:].reshape(RB, FCH, FW)
    v_ref[...] = jnp.transpose(v, (1, 0, 2))


def _proj(x, wcat):
    return pl.pallas_call(
        _proj_body,
        grid=(N // RB,),
        in_specs=[pl.BlockSpec((RB, D_IN), lambda i: (i, 0)),
                  pl.BlockSpec((D_IN, 3 * HID), lambda i: (0, 0))],
        out_specs=[pl.BlockSpec((RB, HID), lambda i: (i, 0)),
                   pl.BlockSpec((RB, HID), lambda i: (i, 0)),
                   pl.BlockSpec((FCH, RB, FW), lambda i: (0, i, 0))],
        out_shape=[jax.ShapeDtypeStruct((N, HID), jnp.float32)] * 2 +
                  [jax.ShapeDtypeStruct((FCH, N, FW), jnp.float32)],
    )(x, wcat)


def _kb_body(k_h, q_h, src3_h, dst3_h, ext_h, den_h,
             srcb, dstb, kr0, kr1, qr0, qr1, exr0, exr1, ext0, ext1, cs, zbuf,
             den_sp, semk0, semk1, semq0, semq1, wse0, wse1, wsd0, wsd1):
    c = lax.axis_index("c")
    s = lax.axis_index("s")
    wid = s * NC + c
    iota = lax.iota(jnp.int32, 16)
    gidx = jnp.minimum(iota, H1 - 1) * 16 + 15
    krs = (kr0, kr1)
    qrs = (qr0, qr1)
    exrs = (exr0, exr1)
    exts = (ext0, ext1)
    semks = (semk0, semk1)
    semqs = (semq0, semq1)
    wses = (wse0, wse1)
    wsds = (wsd0, wsd1)

    def zrow(i, carry):
        zbuf[i, :] = jnp.zeros((16,), jnp.float32)
        return carry
    lax.fori_loop(0, RPS, zrow, 0)
    pltpu.sync_copy(zbuf, den_sp.at[pl.ds(s * RPS, RPS)])
    pltpu.sync_copy(src3_h.at[wid], srcb)
    pltpu.sync_copy(dst3_h.at[wid], dstb)
    plsc.subcore_barrier()

    pltpu.async_copy(k_h.at[srcb.at[0]], kr0, semk0)
    pltpu.async_copy(q_h.at[dstb.at[0]], qr0, semq0)

    def half(b, g):
        kr, qr, exr, ext = krs[b], qrs[b], exrs[b], exts[b]
        nb = 1 - b
        gp = jnp.minimum(g + 1, NCHB - 1)
        pltpu.async_copy(k_h.at[srcb.at[gp]], krs[nb], semks[nb])
        pltpu.async_copy(q_h.at[dstb.at[gp]], qrs[nb], semqs[nb])
        pltpu.make_async_copy(k_h.at[srcb.at[g]], kr, semks[b]).wait()
        pltpu.make_async_copy(q_h.at[dstb.at[g]], qr, semqs[b]).wait()

        @pl.when(g >= 2)
        def _():
            base2 = wid * EW + (g - 2) * CEB
            pltpu.make_async_copy(ext, ext_h.at[:, pl.ds(base2, CEB)], wses[b]).wait()
            pltpu.make_async_copy(exr, den_sp.at[dstb.at[g]], wsds[b]).wait()

        def edge(e, ecarry):
            sums = jnp.zeros((16,), jnp.float32)
            for h in range(H1):
                p = None
                for j in range(4):
                    kv = kr[e, pl.ds(h * D1 + j * 16, 16)]
                    qv = qr[e, pl.ds(h * D1 + j * 16, 16)]
                    t = kv * qv
                    p = t if p is None else p + t
                sums = jnp.where(iota == h, _sum16(p), sums)
            ex = jnp.exp(sums)
            exr[e, :] = ex
            plsc.store_scatter(ext, [iota, jnp.full((16,), e, jnp.int32)], ex)
            return ecarry
        lax.fori_loop(0, CEB, edge, 0, unroll=4)
        base = wid * EW + g * CEB
        pltpu.async_copy(ext, ext_h.at[:, pl.ds(base, CEB)], wses[b])
        pltpu.async_copy(exr, den_sp.at[dstb.at[g]], wsds[b], add=True)

    def chunk(g, carry):
        @pl.when(g % 2 == 0)
        def _():
            half(0, g)

        @pl.when(g % 2 == 1)
        def _():
            half(1, g)
        return carry
    lax.fori_loop(0, NCHB, chunk, 0)

    pltpu.make_async_copy(k_h.at[srcb.at[0]], kr0, semk0).wait()
    pltpu.make_async_copy(q_h.at[dstb.at[0]], qr0, semq0).wait()
    for b in range(2):
        base = wid * EW + (NCHB - 2 + b) * CEB
        pltpu.make_async_copy(exts[b], ext_h.at[:, pl.ds(base, CEB)], wses[b]).wait()
        pltpu.make_async_copy(exrs[b], den_sp.at[dstb.at[0]], wsds[b]).wait()
    plsc.subcore_barrier()
    pltpu.sync_copy(den_sp.at[pl.ds(s * RPS, RPS)],
                    den_h.at[c, pl.ds(s * RPS, RPS)])


_sc_params = pltpu.CompilerParams(needs_layout_passes=False, use_tc_tiling_on_sc=False)

_kb = pl.kernel(
    _kb_body,
    out_type=[jax.ShapeDtypeStruct((16, E), jnp.float32),
              jax.ShapeDtypeStruct((NC, NP, 16), jnp.float32)],
    mesh=_mesh,
    compiler_params=_sc_params,
    scratch_types=[pltpu.VMEM((NCHB, CEB), jnp.int32), pltpu.VMEM((NCHB, CEB), jnp.int32),
                   pltpu.VMEM((CEB, HID), jnp.float32), pltpu.VMEM((CEB, HID), jnp.float32),
                   pltpu.VMEM((CEB, HID), jnp.float32), pltpu.VMEM((CEB, HID), jnp.float32),
                   pltpu.VMEM((CEB, 16), jnp.float32), pltpu.VMEM((CEB, 16), jnp.float32),
                   pltpu.VMEM((16, CEB), jnp.float32), pltpu.VMEM((16, CEB), jnp.float32),
                   pltpu.VMEM((H1 * 16,), jnp.float32),
                   pltpu.VMEM((RPS, 16), jnp.float32),
                   pltpu.VMEM_SHARED((NP, 16), jnp.float32)] +
                  [pltpu.SemaphoreType.DMA] * 8,
    name="hgt_logits",
)


CEA = 400
NCHA = EW // CEA

def _ke_body(v_h, src3_h, dst3_h, ext_h, zeros_h, out_h,
             sidx0, sidx1, dstb, vr0, vr1, msg, exc, acc_sp,
             semv0, semv1, wsm):
    c = lax.axis_index("c")
    s = lax.axis_index("s")
    wid = s * NC + c
    vrs = (vr0, vr1)
    sidxs = (sidx0, sidx1)
    semvs = (semv0, semv1)

    def icopy(t, carry):
        pltpu.sync_copy(dst3_h.at[wid, pl.ds(t * 5, 5)], dstb.at[pl.ds(t * 5, 5)])
        return carry
    lax.fori_loop(0, NCHA // 5, icopy, 0)

    def load_adj_idx(nb, g, fc):
        pltpu.sync_copy(src3_h.at[wid, g], sidxs[nb])
        off = fc * N

        def adj(k, carry):
            sl = pl.ds(k * 16, 16)
            sidxs[nb][sl] = sidxs[nb][sl] + off
            return carry
        lax.fori_loop(0, CEA // 16, adj, 0)

    def fcpass(fc, fcarry):
        pltpu.sync_copy(zeros_h.at[pl.ds(s * RPS, RPS)],
                        acc_sp.at[pl.ds(s * RPS, RPS)])
        plsc.subcore_barrier()

        load_adj_idx(0, 0, fc)
        pltpu.async_copy(v_h.at[sidx0], vr0, semv0)

        def dma_part(b, g):
            nb = 1 - b
            gp = jnp.minimum(g + 1, NCHA - 1)
            load_adj_idx(nb, gp, fc)
            pltpu.async_copy(v_h.at[sidxs[nb]], vrs[nb], semvs[nb])
            base = wid * EW + g * CEA
            pltpu.sync_copy(ext_h.at[fc, pl.ds(base, CEA)], exc)
            pltpu.make_async_copy(v_h.at[sidxs[b]], vrs[b], semvs[b]).wait()

        def calc_part(b, g):
            vr = vrs[b]

            def sub(i, icarry):
                wv = exc[pl.ds(i * 16, 16)]
                for el in range(16):
                    e = i * 16 + el
                    w = _lane_bcast(wv, el)
                    for j in range(FW // 16):
                        msg[e, pl.ds(j * 16, 16)] = vr[e, pl.ds(j * 16, 16)] * w
                return icarry
            lax.fori_loop(0, CEA // 16, sub, 0)

        def chunk(g, carry):
            gc = jnp.minimum(g, NCHA - 1)

            @pl.when(g < NCHA)
            def _():
                @pl.when(g % 2 == 0)
                def _():
                    dma_part(0, g)

                @pl.when(g % 2 == 1)
                def _():
                    dma_part(1, g)

            @pl.when(g >= 1)
            def _():
                pltpu.make_async_copy(msg, acc_sp.at[dstb.at[gc]], wsm).wait()

            @pl.when(g < NCHA)
            def _():
                @pl.when(g % 2 == 0)
                def _():
                    calc_part(0, g)

                @pl.when(g % 2 == 1)
                def _():
                    calc_part(1, g)
                pltpu.async_copy(msg, acc_sp.at[dstb.at[gc]], wsm, add=True)
            return carry
        lax.fori_loop(0, NCHA + 1, chunk, 0)

        pltpu.make_async_copy(v_h.at[sidx1], vr1, semv1).wait()
        plsc.subcore_barrier()
        pltpu.sync_copy(acc_sp.at[pl.ds(s * RPS, RPS)],
                        out_h.at[fc, c, pl.ds(s * RPS, RPS)])
        return fcarry
    lax.fori_loop(0, FCH, fcpass, 0)


_ke = pl.kernel(
    _ke_body,
    out_type=jax.ShapeDtypeStruct((FCH, NC, NP, FW), jnp.float32),
    mesh=_mesh,
    compiler_params=_sc_params,
    scratch_types=[pltpu.VMEM((CEA,), jnp.int32), pltpu.VMEM((CEA,), jnp.int32),
                   pltpu.VMEM((NCHA, CEA), jnp.int32),
                   pltpu.VMEM((CEA, FW), jnp.float32), pltpu.VMEM((CEA, FW), jnp.float32),
                   pltpu.VMEM((CEA, FW), jnp.float32),
                   pltpu.VMEM((CEA,), jnp.float32),
                   pltpu.VMEM_SHARED((NP, FW), jnp.float32)] +
                  [pltpu.SemaphoreType.DMA] * 3,
    name="hgt_agg1",
)


def _kf_body(op_ref, dp_ref, wa_ref, w2_ref, k2_ref, q2_ref, v2_ref):
    dp = dp_ref[...]
    rden = 1.0 / (dp[0] + dp[1] + 1e-16)
    cols = []
    for c in range(FCH):
        part = op_ref[c, 0] + op_ref[c, 1]
        cols.append(part * rden[:, c][:, None])
    agg = jnp.concatenate(cols, axis=1)
    h = jnp.dot(jax.nn.gelu(agg), wa_ref[...], preferred_element_type=jnp.float32)
    kqv = jnp.dot(h, w2_ref[...], preferred_element_type=jnp.float32)
    k2_ref[...] = kqv[:, 0:2]
    q2_ref[...] = kqv[:, 2:4]
    v2_ref[...] = kqv[:, 4:6]


def _kf(outp, denp, wa1, w2cat):
    return pl.pallas_call(
        _kf_body,
        grid=(N // RB,),
        in_specs=[pl.BlockSpec((FCH, NC, RB, FW), lambda i: (0, 0, i, 0)),
                 pl.BlockSpec((NC, RB, 16), lambda i: (0, i, 0)),
                  pl.BlockSpec((HID, HID), lambda i: (0, 0)),
                  pl.BlockSpec((HID, 6), lambda i: (0, 0))],
        out_specs=[pl.BlockSpec((RB, 2), lambda i: (i, 0))] * 3,
        out_shape=[jax.ShapeDtypeStruct((N, 2), jnp.float32)] * 3,
    )(outp, denp, wa1, w2cat)


def _kg_body(k2_h, q2_h, v2_h, src_h, dst3_h, acc_h,
             k2t, q2t, v2t, srcb, dst2, msg0, msg1, zbuf, acc_sp, wsm0, wsm1):
    c = lax.axis_index("c")
    s = lax.axis_index("s")
    wid = s * NC + c
    iota = lax.iota(jnp.int32, 16)
    z16 = jnp.zeros((16,), jnp.int32)
    o16 = jnp.ones((16,), jnp.int32)
    msgs = (msg0, msg1)
    wsms = (wsm0, wsm1)

    pltpu.sync_copy(k2_h, k2t)
    pltpu.sync_copy(q2_h, q2t)
    pltpu.sync_copy(v2_h, v2t)
    pltpu.sync_copy(src_h.at[pl.ds(wid * EW, EW)], srcb)

    def icopy(t, carry):
        pltpu.sync_copy(dst3_h.at[wid, pl.ds(t * 5, 5)], dst2.at[pl.ds(t * 5, 5)])
        return carry
    lax.fori_loop(0, NCH // 5, icopy, 0)

    def zm(i, carry):
        msg0[i, :] = jnp.zeros((16,), jnp.float32)
        msg1[i, :] = jnp.zeros((16,), jnp.float32)
        return carry
    lax.fori_loop(0, CE, zm, 0)

    def zrow(i, carry):
        zbuf[i, :] = jnp.zeros((16,), jnp.float32)
        return carry
    lax.fori_loop(0, RPS, zrow, 0)
    pltpu.sync_copy(zbuf, acc_sp.at[pl.ds(s * RPS, RPS)])
    plsc.subcore_barrier()

    def half(b, g):
        msg = msgs[b]

        @pl.when(g >= 2)
        def _():
            pltpu.make_async_copy(msg, acc_sp.at[dst2.at[g]], wsms[b]).wait()

        def sub(i, icarry):
            sv = srcb[pl.ds(g * CE + i * 16, 16)]
            dv = dst2[g, pl.ds(i * 16, 16)]
            kx = plsc.load_gather(k2t, [z16, sv])
            ky = plsc.load_gather(k2t, [o16, sv])
            qx = plsc.load_gather(q2t, [z16, dv])
            qy = plsc.load_gather(q2t, [o16, dv])
            ex2 = jnp.exp(kx * qx + ky * qy)
            vx = plsc.load_gather(v2t, [z16, sv])
            vy = plsc.load_gather(v2t, [o16, sv])
            rows = i * 16 + iota
            plsc.store_scatter(msg, [rows, z16], ex2)
            plsc.store_scatter(msg, [rows, o16], vx * ex2)
            plsc.store_scatter(msg, [rows, o16 + 1], vy * ex2)
            return icarry
        lax.fori_loop(0, CE // 16, sub, 0)
        pltpu.async_copy(msg, acc_sp.at[dst2.at[g]], wsms[b], add=True)

    def chunk(g, carry):
        @pl.when(g % 2 == 0)
        def _():
            half(0, g)

        @pl.when(g % 2 == 1)
        def _():
            half(1, g)
        return carry
    lax.fori_loop(0, NCH, chunk, 0)
    for b in range(2):
        pltpu.make_async_copy(msgs[b], acc_sp.at[dst2.at[0]], wsms[b]).wait()
    plsc.subcore_barrier()
    pltpu.sync_copy(acc_sp.at[pl.ds(s * RPS, RPS)],
                    acc_h.at[c, pl.ds(s * RPS, RPS)])


_kg = pl.kernel(
    _kg_body,
    out_type=jax.ShapeDtypeStruct((NC, NP, 16), jnp.float32),
    mesh=_mesh,
    compiler_params=_sc_params,
    scratch_types=[pltpu.VMEM((2, N), jnp.float32), pltpu.VMEM((2, N), jnp.float32),
                   pltpu.VMEM((2, N), jnp.float32),
                   pltpu.VMEM((EW,), jnp.int32), pltpu.VMEM((NCH, CE), jnp.int32),
                   pltpu.VMEM((CE, 16), jnp.float32), pltpu.VMEM((CE, 16), jnp.float32),
                   pltpu.VMEM((RPS, 16), jnp.float32),
                   pltpu.VMEM_SHARED((NP, 16), jnp.float32),
                   pltpu.SemaphoreType.DMA, pltpu.SemaphoreType.DMA],
)


def _kh_body(a_ref, wa2_ref, o_ref):
    a = a_ref[0] + a_ref[1]
    den = a[:, 0:1]
    num = a[:, 1:3]
    o_ref[...] = jnp.dot(jax.nn.gelu(num / (den + 1e-16)), wa2_ref[...],
                         preferred_element_type=jnp.float32)


def _kh(acc2, wa2):
    return pl.pallas_call(
        _kh_body,
        grid=(N // RB,),
        in_specs=[pl.BlockSpec((NC, RB, 16), lambda i: (0, i, 0)),
                  pl.BlockSpec((2, 2), lambda i: (0, 0))],
        out_specs=pl.BlockSpec((RB, 2), lambda i: (i, 0)),
        out_shape=jax.ShapeDtypeStruct((N, 2), jnp.float32),
    )(acc2, wa2)


def kernel(x, edge_index, Wk1, Wq1, Wv1, a_rel1, m_rel1, pri1, Wa1,
           Wk2, Wq2, Wv2, a_rel2, m_rel2, pri2, Wa2):
    ei = edge_index.astype(jnp.int32)
    src, dst = ei[0], ei[1]

    scale1 = pri1 / jnp.sqrt(jnp.float32(D1))
    wk1e = jnp.einsum("ihd,hde->ihe", Wk1.reshape(D_IN, H1, D1), a_rel1)
    wq1s = Wq1.reshape(D_IN, H1, D1) * scale1[None, :, None]
    wv1e = jnp.einsum("ihd,hde->ihe", Wv1.reshape(D_IN, H1, D1), m_rel1)
    wcat = jnp.concatenate([wk1e.reshape(D_IN, HID), wq1s.reshape(D_IN, HID),
                            wv1e.reshape(D_IN, HID)], axis=1)

    k1, q1, vstk = _proj(x, wcat)
    src3b = src.reshape(NW, NCHB, CEB)
    dst3b = dst.reshape(NW, NCHB, CEB)
    src3e = src.reshape(NW, NCHA, CEA)
    dst3e = dst.reshape(NW, NCHA, CEA)
    ext, denp = _kb(k1, q1, src3b, dst3b)
    zeros_big = jnp.zeros((NP, FW), jnp.float32)
    vcat = vstk.reshape(FCH * N, FW)
    outp = _ke(vcat, src3e, dst3e, ext, zeros_big)

    d2 = OUT
    w2k = Wk2 @ a_rel2[0]
    w2q = Wq2 * (pri2[0] / jnp.sqrt(jnp.float32(d2)))
    w2v = Wv2 @ m_rel2[0]
    w2cat = jnp.concatenate([w2k, w2q, w2v], axis=1)

    k2, q2, v2n = _kf(outp, denp, Wa1, w2cat)
    dst3g = dst.reshape(NW, NCH, CE)
    acc2 = _kg(k2.T, q2.T, v2n.T, src, dst3g)
    return _kh(acc2, Wa2)
